# baseline (speedup 1.0000x reference)
import functools

import jax
import jax.numpy as jnp
from jax import lax
from jax.experimental import pallas as pl
from jax.experimental.pallas import tpu as pltpu
from jax.experimental.pallas import tpu_sc as plsc

VOCAB = 1000000
DIM = 64
N = 4096 * 200
NUM_WORKERS = 32
PER_WORKER = N // NUM_WORKERS
CHUNK = 128
NCHUNKS = PER_WORKER // CHUNK


@functools.partial(
    pl.kernel,
    mesh=plsc.VectorSubcoreMesh(core_axis_name="c", subcore_axis_name="s"),
    out_type=jax.ShapeDtypeStruct((N, DIM), jnp.float32),
    scratch_types=[
        pltpu.VMEM((CHUNK,), jnp.int32),
        pltpu.VMEM((CHUNK, DIM), jnp.float32),
        pltpu.SemaphoreType.DMA,
    ],
    compiler_params=pltpu.CompilerParams(use_tc_tiling_on_sc=False),
)
def _gather_kernel(idx_hbm, table_hbm, out_hbm, idx_v, rows_v, sem):
    wid = lax.axis_index("s") * 2 + lax.axis_index("c")
    base = wid * PER_WORKER

    def step(j, carry):
        off = base + j * CHUNK
        pltpu.sync_copy(idx_hbm.at[pl.ds(off, CHUNK)], idx_v)
        pltpu.async_copy(table_hbm.at[idx_v], rows_v, sem).wait()
        pltpu.sync_copy(rows_v, out_hbm.at[pl.ds(off, CHUNK)])
        return carry

    lax.fori_loop(0, NCHUNKS, step, 0)


def kernel(tokens, emb_weight):
    idx_flat = tokens.reshape(-1).astype(jnp.int32)
    out = _gather_kernel(idx_flat, emb_weight)
    return out.reshape(tokens.shape[0], tokens.shape[1], DIM)


# speedup vs baseline: 1.1905x; 1.1905x over previous
import functools

import jax
import jax.numpy as jnp
from jax import lax
from jax.experimental import pallas as pl
from jax.experimental.pallas import tpu as pltpu
from jax.experimental.pallas import tpu_sc as plsc

VOCAB = 1000000
DIM = 64
N = 4096 * 200
NUM_WORKERS = 32
PER_WORKER = N // NUM_WORKERS
CHUNK = 128
NCHUNKS = PER_WORKER // CHUNK
NBUF = 8
AHEAD = 5
GROUPS = NCHUNKS // NBUF


@functools.partial(
    pl.kernel,
    mesh=plsc.VectorSubcoreMesh(core_axis_name="c", subcore_axis_name="s"),
    out_type=jax.ShapeDtypeStruct((N, DIM), jnp.float32),
    scratch_types=[
        pltpu.VMEM((PER_WORKER,), jnp.int32),
        tuple(pltpu.VMEM((CHUNK, DIM), jnp.float32) for _ in range(NBUF)),
        tuple(pltpu.SemaphoreType.DMA for _ in range(NBUF)),
        tuple(pltpu.SemaphoreType.DMA for _ in range(NBUF)),
    ],
    compiler_params=pltpu.CompilerParams(use_tc_tiling_on_sc=False),
)
def _gather_kernel(idx_hbm, table_hbm, out_hbm, idx_v, rows, gsem, osem):
    wid = lax.axis_index("s") * 2 + lax.axis_index("c")
    base = wid * PER_WORKER

    pltpu.sync_copy(idx_hbm.at[pl.ds(base, PER_WORKER)], idx_v)

    def fire_gather(j, b):
        pltpu.async_copy(
            table_hbm.at[idx_v.at[pl.ds(j * CHUNK, CHUNK)]], rows[b], gsem[b]
        )

    for j in range(AHEAD):
        fire_gather(j, j % NBUF)

    def group(g, carry):
        for b in range(NBUF):
            j = g * NBUF + b
            jp = j + AHEAD
            bp = (b + AHEAD) % NBUF

            @pl.when(jnp.logical_and(jp >= NBUF, jp < NCHUNKS))
            def _():
                pltpu.make_async_copy(
                    rows[bp], out_hbm.at[pl.ds(base, CHUNK)], osem[bp]
                ).wait()

            @pl.when(jp < NCHUNKS)
            def _():
                fire_gather(jp, bp)

            pltpu.make_async_copy(
                table_hbm.at[idx_v.at[pl.ds(j * CHUNK, CHUNK)]], rows[b], gsem[b]
            ).wait()
            pltpu.async_copy(
                rows[b], out_hbm.at[pl.ds(base + j * CHUNK, CHUNK)], osem[b]
            )
        return carry

    lax.fori_loop(0, GROUPS, group, 0)

    for b in range(NBUF):
        pltpu.make_async_copy(
            rows[b], out_hbm.at[pl.ds(base, CHUNK)], osem[b]
        ).wait()


def kernel(tokens, emb_weight):
    idx_flat = tokens.reshape(-1).astype(jnp.int32)
    out = _gather_kernel(idx_flat, emb_weight)
    return out.reshape(tokens.shape[0], tokens.shape[1], DIM)


# speedup vs baseline: 1.5895x; 1.3352x over previous
import functools

import jax
import jax.numpy as jnp
from jax import lax
from jax.experimental import pallas as pl
from jax.experimental.pallas import tpu as pltpu
from jax.experimental.pallas import tpu_sc as plsc

VOCAB = 1000000
DIM = 64
PDIM = 128
N = 4096 * 200
NUM_WORKERS = 32
PER_WORKER = N // NUM_WORKERS
CHUNK = 128
NCHUNKS = PER_WORKER // CHUNK
NBUF = 8
AHEAD = 5
GROUPS = NCHUNKS // NBUF


@functools.partial(
    pl.kernel,
    mesh=plsc.VectorSubcoreMesh(core_axis_name="c", subcore_axis_name="s"),
    out_type=jax.ShapeDtypeStruct((N, PDIM), jnp.float32),
    scratch_types=[
        pltpu.VMEM((PER_WORKER,), jnp.int32),
        tuple(pltpu.VMEM((CHUNK, DIM), jnp.float32) for _ in range(NBUF)),
        tuple(pltpu.SemaphoreType.DMA for _ in range(NBUF)),
        tuple(pltpu.SemaphoreType.DMA for _ in range(NBUF)),
    ],
    compiler_params=pltpu.CompilerParams(use_tc_tiling_on_sc=False),
)
def _gather_kernel(idx_hbm, table_hbm, out_hbm, idx_v, rows, gsem, osem):
    wid = lax.axis_index("s") * 2 + lax.axis_index("c")
    base = wid * PER_WORKER

    pltpu.sync_copy(idx_hbm.at[pl.ds(base, PER_WORKER)], idx_v)

    def fire_gather(j, b):
        pltpu.async_copy(
            table_hbm.at[idx_v.at[pl.ds(j * CHUNK, CHUNK)]], rows[b], gsem[b]
        )

    for j in range(AHEAD):
        fire_gather(j, j % NBUF)

    def group(g, carry):
        for b in range(NBUF):
            j = g * NBUF + b
            jp = j + AHEAD
            bp = (b + AHEAD) % NBUF

            @pl.when(jnp.logical_and(jp >= NBUF, jp < NCHUNKS))
            def _():
                pltpu.make_async_copy(
                    rows[bp],
                    out_hbm.at[pl.ds(base, CHUNK), pl.ds(0, DIM)],
                    osem[bp],
                ).wait()

            @pl.when(jp < NCHUNKS)
            def _():
                fire_gather(jp, bp)

            pltpu.make_async_copy(
                table_hbm.at[idx_v.at[pl.ds(j * CHUNK, CHUNK)]], rows[b], gsem[b]
            ).wait()
            pltpu.async_copy(
                rows[b],
                out_hbm.at[pl.ds(base + j * CHUNK, CHUNK), pl.ds(0, DIM)],
                osem[b],
            )
        return carry

    lax.fori_loop(0, GROUPS, group, 0)

    for b in range(NBUF):
        pltpu.make_async_copy(
            rows[b], out_hbm.at[pl.ds(base, CHUNK), pl.ds(0, DIM)], osem[b]
        ).wait()


def kernel(tokens, emb_weight):
    idx_flat = tokens.reshape(-1).astype(jnp.int32)
    out = _gather_kernel(idx_flat, emb_weight)
    out3 = out.reshape(tokens.shape[0], tokens.shape[1], PDIM)
    return out3[:, :, :DIM]
